# Initial kernel scaffold; baseline (speedup 1.0000x reference)
import jax
import jax.numpy as jnp
from jax.experimental import pallas as pl


def kernel(features, center_index, edge_index, perm, W1, b1, W2, b2, Wd):
    raise NotImplementedError("write your pallas kernel here")


# speedup vs baseline: 6.5252x; 6.5252x over previous
import functools

import jax
import jax.numpy as jnp
from jax import lax
from jax.experimental import pallas as pl
from jax.experimental.pallas import tpu as pltpu
from jax.experimental.pallas import tpu_sc as plsc

N = 10000
E = 160000
FIN = 256
HID = 512
K = 64
BETA = 1.0
ALPHA = 0.5

CW = 128
BB = 80
NBATCH = E // BB
NSUB = 16
BPS = NBATCH // NSUB
RPS = N // NSUB
RB = 10
RBS = N // RB

_mesh_cache = []


def _mesh():
  if not _mesh_cache:
    _mesh_cache.append(
        plsc.VectorSubcoreMesh(core_axis_name="c", subcore_axis_name="s",
                               num_cores=2, num_subcores=16))
  return _mesh_cache[0]


def _sc1_call(src2d, dst2d, perm2d, features, ones8, zeros1, interpret=False):
  def body(src_ref, dst_ref, perm_ref, f_ref, ones_hbm, zer_hbm,
           cs_out, cd_out, fp_out,
           acc_s, acc_d, sidx, didx, pidx, ones_v, rows_v, sem):
    c = lax.axis_index("c")
    s = lax.axis_index("s")

    @pl.when(c == 0)
    def _():
      pltpu.sync_copy(zer_hbm, acc_s.at[pl.ds(s * RPS, RPS)])
      pltpu.sync_copy(zer_hbm, acc_d.at[pl.ds(s * RPS, RPS)])
      pltpu.sync_copy(ones_hbm, ones_v)
      pltpu.sync_copy(src_ref.at[pl.ds(s * BPS, BPS)], sidx)
      pltpu.sync_copy(dst_ref.at[pl.ds(s * BPS, BPS)], didx)
      plsc.subcore_barrier()

      def bb(j, carry):
        pltpu.sync_copy(ones_v, acc_s.at[sidx.at[j]], add=True)
        pltpu.sync_copy(ones_v, acc_d.at[didx.at[j]], add=True)
        return carry

      lax.fori_loop(0, BPS, bb, 0)
      plsc.subcore_barrier()
      pltpu.sync_copy(acc_s.at[pl.ds(s * RPS, RPS)],
                      cs_out.at[pl.ds(s * RPS, RPS)])
      pltpu.sync_copy(acc_d.at[pl.ds(s * RPS, RPS)],
                      cd_out.at[pl.ds(s * RPS, RPS)])

    @pl.when(c == 1)
    def _():
      nrow_batch = N // BB

      def fb(it, carry):
        b = s + it * NSUB

        @pl.when(b < nrow_batch)
        def _():
          pltpu.sync_copy(perm_ref.at[pl.ds(b, 1)], pidx)
          pltpu.async_copy(f_ref.at[pidx.at[0]], rows_v, sem).wait()
          pltpu.sync_copy(rows_v, fp_out.at[pl.ds(b * BB, BB)])

        return carry

      lax.fori_loop(0, 8, fb, 0)

  f = pl.kernel(
      body,
      out_type=[
          jax.ShapeDtypeStruct((N, 8), jnp.float32),
          jax.ShapeDtypeStruct((N, 8), jnp.float32),
          jax.ShapeDtypeStruct((N, FIN), jnp.float32),
      ],
      mesh=_mesh(),
      compiler_params=pltpu.CompilerParams(use_tc_tiling_on_sc=False),
      scratch_types=[
          pltpu.VMEM_SHARED((N, 8), jnp.float32),
          pltpu.VMEM_SHARED((N, 8), jnp.float32),
          pltpu.VMEM((BPS, BB), jnp.int32),
          pltpu.VMEM((BPS, BB), jnp.int32),
          pltpu.VMEM((1, BB), jnp.int32),
          pltpu.VMEM((BB, 8), jnp.float32),
          pltpu.VMEM((BB, FIN), jnp.float32),
          pltpu.SemaphoreType.DMA,
      ],
      interpret=interpret,
  )
  return f(src2d, dst2d, perm2d, features, ones8, zeros1)


def _segsum_call(src2d, dst2d, tables, zeros_z, interpret=False):
  nchunk = len(tables)
  ncpc = nchunk // 2

  def body(src_ref, dst_ref, *rest):
    tabs = rest[:nchunk]
    zer_hbm = rest[nchunk]
    outs = rest[nchunk + 1:2 * nchunk + 1]
    acc, sidx, didx, rows_v, sem = rest[2 * nchunk + 1:]
    c = lax.axis_index("c")
    s = lax.axis_index("s")
    pltpu.sync_copy(src_ref.at[pl.ds(s * BPS, BPS)], sidx)
    pltpu.sync_copy(dst_ref.at[pl.ds(s * BPS, BPS)], didx)

    def one_chunk(tbl, out):
      pltpu.sync_copy(zer_hbm, acc.at[pl.ds(s * RPS, RPS)])
      plsc.subcore_barrier()

      def bb(j, carry):
        pltpu.async_copy(tbl.at[sidx.at[j]], rows_v, sem).wait()
        pltpu.sync_copy(rows_v, acc.at[didx.at[j]], add=True)
        return carry

      lax.fori_loop(0, BPS, bb, 0)
      plsc.subcore_barrier()
      pltpu.sync_copy(acc.at[pl.ds(s * RPS, RPS)],
                      out.at[pl.ds(s * RPS, RPS)])

    for ci in range(ncpc):
      for half in range(2):
        idx = half * ncpc + ci

        @pl.when(c == half)
        def _(idx=idx):
          one_chunk(tabs[idx], outs[idx])

  f = pl.kernel(
      body,
      out_type=[jax.ShapeDtypeStruct((N, CW), jnp.float32)
                for _ in range(nchunk)],
      mesh=_mesh(),
      compiler_params=pltpu.CompilerParams(use_tc_tiling_on_sc=False),
      scratch_types=[
          pltpu.VMEM_SHARED((N, CW), jnp.float32),
          pltpu.VMEM((BPS, BB), jnp.int32),
          pltpu.VMEM((BPS, BB), jnp.int32),
          pltpu.VMEM((BB, CW), jnp.float32),
          pltpu.SemaphoreType.DMA,
      ],
      interpret=interpret,
  )
  return f(src2d, dst2d, *tables, zeros_z)


def _tc2_call(features, fp, cnt_s, cnt_d, interpret=False):
  def body(f_ref, fp_ref, cs_ref, cd_ref, g_ref, din_ref, ta, tb, tc, td):
    deg_o = jnp.maximum(cs_ref[:, :1], 1.0)
    deg_i = jnp.maximum(cd_ref[:, :1], 1.0)
    g = lax.rsqrt(deg_o)
    din = lax.rsqrt(deg_i)
    g_ref[...] = g
    din_ref[...] = din
    xp = f_ref[...] * g
    xn = fp_ref[...] * g
    ta[...] = xp[:, :CW]
    tb[...] = xp[:, CW:]
    tc[...] = xn[:, :CW]
    td[...] = xn[:, CW:]

  return pl.pallas_call(
      body,
      grid=(RB,),
      in_specs=[
          pl.BlockSpec((RBS, FIN), lambda i: (i, 0)),
          pl.BlockSpec((RBS, FIN), lambda i: (i, 0)),
          pl.BlockSpec((RBS, 8), lambda i: (i, 0)),
          pl.BlockSpec((RBS, 8), lambda i: (i, 0)),
      ],
      out_specs=[
          pl.BlockSpec((RBS, 1), lambda i: (i, 0)),
          pl.BlockSpec((RBS, 1), lambda i: (i, 0)),
          pl.BlockSpec((RBS, CW), lambda i: (i, 0)),
          pl.BlockSpec((RBS, CW), lambda i: (i, 0)),
          pl.BlockSpec((RBS, CW), lambda i: (i, 0)),
          pl.BlockSpec((RBS, CW), lambda i: (i, 0)),
      ],
      out_shape=[
          jax.ShapeDtypeStruct((N, 1), jnp.float32),
          jax.ShapeDtypeStruct((N, 1), jnp.float32),
          jax.ShapeDtypeStruct((N, CW), jnp.float32),
          jax.ShapeDtypeStruct((N, CW), jnp.float32),
          jax.ShapeDtypeStruct((N, CW), jnp.float32),
          jax.ShapeDtypeStruct((N, CW), jnp.float32),
      ],
      interpret=interpret,
  )(features, fp, cnt_s, cnt_d)


def _tc3_call(s1, g, din, W1, b1, interpret=False):
  def body(sa, sb, sc, sd, g_ref, din_ref, w_ref, b_ref, *outs):
    din_b = din_ref[...]
    g_b = g_ref[...]
    b = b_ref[...]
    pos = (jnp.dot(sa[...] * din_b, w_ref[:CW, :],
                   preferred_element_type=jnp.float32)
           + jnp.dot(sb[...] * din_b, w_ref[CW:, :],
                     preferred_element_type=jnp.float32) + b)
    neg = (jnp.dot(sc[...] * din_b, w_ref[:CW, :],
                   preferred_element_type=jnp.float32)
           + jnp.dot(sd[...] * din_b, w_ref[CW:, :],
                     preferred_element_type=jnp.float32) + b)
    rp = jnp.maximum(pos, 0.0) * g_b
    rn = jnp.maximum(neg, 0.0) * g_b
    for k in range(4):
      outs[k][...] = rp[:, k * CW:(k + 1) * CW]
      outs[4 + k][...] = rn[:, k * CW:(k + 1) * CW]

  return pl.pallas_call(
      body,
      grid=(RB,),
      in_specs=[
          pl.BlockSpec((RBS, CW), lambda i: (i, 0)),
          pl.BlockSpec((RBS, CW), lambda i: (i, 0)),
          pl.BlockSpec((RBS, CW), lambda i: (i, 0)),
          pl.BlockSpec((RBS, CW), lambda i: (i, 0)),
          pl.BlockSpec((RBS, 1), lambda i: (i, 0)),
          pl.BlockSpec((RBS, 1), lambda i: (i, 0)),
          pl.BlockSpec((FIN, HID), lambda i: (0, 0)),
          pl.BlockSpec((1, HID), lambda i: (0, 0)),
      ],
      out_specs=[pl.BlockSpec((RBS, CW), lambda i: (i, 0))
                 for _ in range(8)],
      out_shape=[jax.ShapeDtypeStruct((N, CW), jnp.float32)
                 for _ in range(8)],
      interpret=interpret,
  )(*s1, g, din, W1, b1)


def _tc4_call(s2, din, W2, b2, ci, interpret=False):
  def body(c0, c1, c2, c3, c4, c5, c6, c7, din_ref, w_ref, b_ref, ci_ref,
           pos_out, neg_out, sum_out, mu_out):
    i = pl.program_id(0)
    din_b = din_ref[...]
    chunks = (c0, c1, c2, c3, c4, c5, c6, c7)
    pos = b_ref[...]
    neg = b_ref[...]
    for k in range(4):
      pos = pos + jnp.dot(chunks[k][...] * din_b,
                          w_ref[k * CW:(k + 1) * CW, :],
                          preferred_element_type=jnp.float32)
      neg = neg + jnp.dot(chunks[4 + k][...] * din_b,
                          w_ref[k * CW:(k + 1) * CW, :],
                          preferred_element_type=jnp.float32)
    pos_out[...] = pos
    neg_out[...] = neg
    nrm = jnp.sqrt(jnp.sum(pos * pos, axis=1, keepdims=True))
    h1 = pos / (nrm + 1e-6)
    rows = i * RBS + lax.broadcasted_iota(jnp.int32, (K, RBS), 1)
    oh = (ci_ref[...] == rows).astype(jnp.float32)
    mu_part = jnp.dot(oh, h1, preferred_element_type=jnp.float32)
    sp = jnp.sum(pos, axis=0, keepdims=True)

    @pl.when(i == 0)
    def _():
      sum_out[...] = sp
      mu_out[...] = mu_part

    @pl.when(i > 0)
    def _():
      sum_out[...] += sp
      mu_out[...] += mu_part

  return pl.pallas_call(
      body,
      grid=(RB,),
      in_specs=[pl.BlockSpec((RBS, CW), lambda i: (i, 0))
                for _ in range(8)] + [
          pl.BlockSpec((RBS, 1), lambda i: (i, 0)),
          pl.BlockSpec((HID, HID), lambda i: (0, 0)),
          pl.BlockSpec((1, HID), lambda i: (0, 0)),
          pl.BlockSpec((K, 1), lambda i: (0, 0)),
      ],
      out_specs=[
          pl.BlockSpec((RBS, HID), lambda i: (i, 0)),
          pl.BlockSpec((RBS, HID), lambda i: (i, 0)),
          pl.BlockSpec((1, HID), lambda i: (0, 0)),
          pl.BlockSpec((K, HID), lambda i: (0, 0)),
      ],
      out_shape=[
          jax.ShapeDtypeStruct((N, HID), jnp.float32),
          jax.ShapeDtypeStruct((N, HID), jnp.float32),
          jax.ShapeDtypeStruct((1, HID), jnp.float32),
          jax.ShapeDtypeStruct((K, HID), jnp.float32),
      ],
      interpret=interpret,
  )(*s2, din, W2, b2, ci)


def _tc5_call(sum_pos, mu_raw, Wd, interpret=False):
  def body(s_ref, m_ref, wd_ref, v_out, mu_out):
    gs = jax.nn.sigmoid(s_ref[...] / N)
    v_out[...] = lax.dot_general(gs, wd_ref[...], (((1,), (1,)), ((), ())),
                                 preferred_element_type=jnp.float32)
    m = m_ref[...]
    mu_out[...] = m / (jnp.sqrt(jnp.sum(m * m, axis=1, keepdims=True)) + 1e-6)

  return pl.pallas_call(
      body,
      out_shape=[
          jax.ShapeDtypeStruct((1, HID), jnp.float32),
          jax.ShapeDtypeStruct((K, HID), jnp.float32),
      ],
      interpret=interpret,
  )(sum_pos, mu_raw, Wd)


def _tc6_call(pos, neg, v, mu, interpret=False):
  def body(pos_ref, neg_ref, v_ref, mu_ref, out_ref):
    i = pl.program_id(0)
    p = pos_ref[...]
    n = neg_ref[...]
    v = v_ref[...]
    mu = mu_ref[...]
    pg = lax.dot_general(p, v, (((1,), (1,)), ((), ())),
                         preferred_element_type=jnp.float32)
    ng = lax.dot_general(n, v, (((1,), (1,)), ((), ())),
                         preferred_element_type=jnp.float32)
    nrm = jnp.sqrt(jnp.sum(p * p, axis=1, keepdims=True))
    h1 = p / (nrm + 1e-6)
    dist = lax.dot_general(h1, mu, (((1,), (1,)), ((), ())),
                           preferred_element_type=jnp.float32)
    z = BETA * dist
    z = z - jnp.max(z, axis=1, keepdims=True)
    ez = jnp.exp(z)
    r = ez / jnp.sum(ez, axis=1, keepdims=True)
    cs = jax.nn.sigmoid(jnp.dot(r, mu, preferred_element_type=jnp.float32))
    pc = jnp.sum(p * cs, axis=1, keepdims=True)
    nc = jnp.sum(n * cs, axis=1, keepdims=True)

    def bce_sum(x, target_one):
      sp = jnp.log1p(jnp.exp(-jnp.abs(x)))
      t = jnp.maximum(x, 0.0) + sp
      if target_one:
        t = t - x
      return jnp.sum(t)

    contrib = (bce_sum(pg, True) + bce_sum(ng, False)
               + ALPHA * (bce_sum(pc, True) + bce_sum(nc, False))) / N

    contrib2d = jnp.reshape(contrib, (1, 1))

    @pl.when(i == 0)
    def _():
      out_ref[...] = contrib2d

    @pl.when(i > 0)
    def _():
      out_ref[...] += contrib2d

  return pl.pallas_call(
      body,
      grid=(RB,),
      in_specs=[
          pl.BlockSpec((RBS, HID), lambda i: (i, 0)),
          pl.BlockSpec((RBS, HID), lambda i: (i, 0)),
          pl.BlockSpec((1, HID), lambda i: (0, 0)),
          pl.BlockSpec((K, HID), lambda i: (0, 0)),
      ],
      out_specs=pl.BlockSpec((1, 1), lambda i: (0, 0)),
      out_shape=jax.ShapeDtypeStruct((1, 1), jnp.float32),
      interpret=interpret,
  )(pos, neg, v, mu)


def _kernel_impl(features, center_index, edge_index, perm, W1, b1, W2, b2, Wd,
                 interpret=False):
  src2d = edge_index[0].reshape(NBATCH, BB).astype(jnp.int32)
  dst2d = edge_index[1].reshape(NBATCH, BB).astype(jnp.int32)
  perm2d = perm.reshape(N // BB, BB).astype(jnp.int32)
  ones8 = jnp.ones((BB, 8), jnp.float32)
  zeros1 = jnp.zeros((RPS, 8), jnp.float32)
  zeros_z = jnp.zeros((RPS, CW), jnp.float32)

  cnt_s, cnt_d, fp = _sc1_call(src2d, dst2d, perm2d, features, ones8, zeros1,
                               interpret=interpret)
  g, din, ta, tb, tc, td = _tc2_call(features, fp, cnt_s, cnt_d,
                                     interpret=interpret)
  s1 = _segsum_call(src2d, dst2d, (ta, tb, tc, td), zeros_z,
                    interpret=interpret)
  r8 = _tc3_call(s1, g, din, W1, b1.reshape(1, HID), interpret=interpret)
  s2 = _segsum_call(src2d, dst2d, r8, zeros_z, interpret=interpret)
  pos, neg, sum_pos, mu_raw = _tc4_call(
      s2, din, W2, b2.reshape(1, HID),
      center_index.reshape(K, 1).astype(jnp.int32), interpret=interpret)
  v, mu = _tc5_call(sum_pos, mu_raw, Wd, interpret=interpret)
  out = _tc6_call(pos, neg, v, mu, interpret=interpret)
  return out[0, 0]


def kernel(features, center_index, edge_index, perm, W1, b1, W2, b2, Wd):
  return _kernel_impl(features, center_index, edge_index, perm,
                      W1, b1, W2, b2, Wd)


# speedup vs baseline: 7.3366x; 1.1243x over previous
import functools

import jax
import jax.numpy as jnp
from jax import lax
from jax.experimental import pallas as pl
from jax.experimental.pallas import tpu as pltpu
from jax.experimental.pallas import tpu_sc as plsc

N = 10000
E = 160000
FIN = 256
HID = 512
K = 64
BETA = 1.0
ALPHA = 0.5

CW = 128
BB = 80
NBATCH = E // BB
NSUB = 16
BPS = NBATCH // NSUB
BPS2 = 126
NBATCH2 = BPS2 * NSUB
E2 = NBATCH2 * BB
NA = N + 8
RPS = N // NSUB
RB = 10
RBS = N // RB

_mesh_cache = []


def _mesh():
  if not _mesh_cache:
    _mesh_cache.append(
        plsc.VectorSubcoreMesh(core_axis_name="c", subcore_axis_name="s",
                               num_cores=2, num_subcores=16))
  return _mesh_cache[0]


def _sc1_call(src2d, dst2d, perm2d, features, ones8, zeros1, interpret=False):
  def body(src_ref, dst_ref, perm_ref, f_ref, ones_hbm, zer_hbm,
           cs_out, cd_out, fp_out,
           acc_s, acc_d, sidx, didx, pidx, ones_v, rows_v, sem):
    c = lax.axis_index("c")
    s = lax.axis_index("s")

    @pl.when(c == 0)
    def _():
      pltpu.sync_copy(zer_hbm, acc_s.at[pl.ds(s * RPS, RPS)])
      pltpu.sync_copy(zer_hbm, acc_d.at[pl.ds(s * RPS, RPS)])
      pltpu.sync_copy(ones_hbm, ones_v)
      pltpu.sync_copy(src_ref.at[pl.ds(s * BPS, BPS)], sidx)
      pltpu.sync_copy(dst_ref.at[pl.ds(s * BPS, BPS)], didx)
      plsc.subcore_barrier()

      def bb(j, carry):
        pltpu.sync_copy(ones_v, acc_s.at[sidx.at[j]], add=True)
        pltpu.sync_copy(ones_v, acc_d.at[didx.at[j]], add=True)
        return carry

      lax.fori_loop(0, BPS, bb, 0)
      plsc.subcore_barrier()
      pltpu.sync_copy(acc_s.at[pl.ds(s * RPS, RPS)],
                      cs_out.at[pl.ds(s * RPS, RPS)])
      pltpu.sync_copy(acc_d.at[pl.ds(s * RPS, RPS)],
                      cd_out.at[pl.ds(s * RPS, RPS)])

    @pl.when(c == 1)
    def _():
      nrow_batch = N // BB

      def fb(it, carry):
        b = s + it * NSUB

        @pl.when(b < nrow_batch)
        def _():
          pltpu.sync_copy(perm_ref.at[pl.ds(b, 1)], pidx)
          pltpu.async_copy(f_ref.at[pidx.at[0]], rows_v, sem).wait()
          pltpu.sync_copy(rows_v, fp_out.at[pl.ds(b * BB, BB)])

        return carry

      lax.fori_loop(0, 8, fb, 0)

  f = pl.kernel(
      body,
      out_type=[
          jax.ShapeDtypeStruct((N, 8), jnp.float32),
          jax.ShapeDtypeStruct((N, 8), jnp.float32),
          jax.ShapeDtypeStruct((N, FIN), jnp.float32),
      ],
      mesh=_mesh(),
      compiler_params=pltpu.CompilerParams(use_tc_tiling_on_sc=False),
      scratch_types=[
          pltpu.VMEM_SHARED((N, 8), jnp.float32),
          pltpu.VMEM_SHARED((N, 8), jnp.float32),
          pltpu.VMEM((BPS, BB), jnp.int32),
          pltpu.VMEM((BPS, BB), jnp.int32),
          pltpu.VMEM((1, BB), jnp.int32),
          pltpu.VMEM((BB, 8), jnp.float32),
          pltpu.VMEM((BB, FIN), jnp.float32),
          pltpu.SemaphoreType.DMA,
      ],
      interpret=interpret,
  )
  return f(src2d, dst2d, perm2d, features, ones8, zeros1)


def _segsum_call(src2d, dst2d, tables, zeros_z, interpret=False):
  nchunk = len(tables)
  ncpc = nchunk // 2

  def body(src_ref, dst_ref, *rest):
    tabs = rest[:nchunk]
    zer_hbm = rest[nchunk]
    outs = rest[nchunk + 1:2 * nchunk + 1]
    acc, sidx, didx, rows0, rows1, sem0, sem1 = rest[2 * nchunk + 1:]
    c = lax.axis_index("c")
    s = lax.axis_index("s")
    pltpu.sync_copy(src_ref.at[pl.ds(s * BPS2, BPS2)], sidx)
    pltpu.sync_copy(dst_ref.at[pl.ds(s * BPS2, BPS2)], didx)

    def one_chunk(tbl, out):
      pltpu.sync_copy(zer_hbm, acc.at[pl.ds(s * RPS, RPS)])
      plsc.subcore_barrier()
      pltpu.async_copy(tbl.at[sidx.at[0]], rows0, sem0)

      def bb(j2, carry):
        j = 2 * j2
        pltpu.async_copy(tbl.at[sidx.at[j + 1]], rows1, sem1)
        pltpu.make_async_copy(tbl.at[sidx.at[j]], rows0, sem0).wait()
        pltpu.sync_copy(rows0, acc.at[didx.at[j]], add=True)

        @pl.when(j2 < BPS2 // 2 - 1)
        def _():
          pltpu.async_copy(tbl.at[sidx.at[j + 2]], rows0, sem0)

        pltpu.make_async_copy(tbl.at[sidx.at[j + 1]], rows1, sem1).wait()
        pltpu.sync_copy(rows1, acc.at[didx.at[j + 1]], add=True)
        return carry

      lax.fori_loop(0, BPS2 // 2, bb, 0)
      plsc.subcore_barrier()
      pltpu.sync_copy(acc.at[pl.ds(s * RPS, RPS)],
                      out.at[pl.ds(s * RPS, RPS)])

    for ci in range(ncpc):
      for half in range(2):
        idx = half * ncpc + ci

        @pl.when(c == half)
        def _(idx=idx):
          one_chunk(tabs[idx], outs[idx])

  f = pl.kernel(
      body,
      out_type=[jax.ShapeDtypeStruct((N, CW), jnp.float32)
                for _ in range(nchunk)],
      mesh=_mesh(),
      compiler_params=pltpu.CompilerParams(use_tc_tiling_on_sc=False),
      scratch_types=[
          pltpu.VMEM_SHARED((NA, CW), jnp.float32),
          pltpu.VMEM((BPS2, BB), jnp.int32),
          pltpu.VMEM((BPS2, BB), jnp.int32),
          pltpu.VMEM((BB, CW), jnp.float32),
          pltpu.VMEM((BB, CW), jnp.float32),
          pltpu.SemaphoreType.DMA,
          pltpu.SemaphoreType.DMA,
      ],
      interpret=interpret,
  )
  return f(src2d, dst2d, *tables, zeros_z)


def _tc2_call(features, fp, cnt_s, cnt_d, interpret=False):
  def body(f_ref, fp_ref, cs_ref, cd_ref, g_ref, din_ref, ta, tb, tc, td):
    deg_o = jnp.maximum(cs_ref[:, :1], 1.0)
    deg_i = jnp.maximum(cd_ref[:, :1], 1.0)
    g = lax.rsqrt(deg_o)
    din = lax.rsqrt(deg_i)
    g_ref[...] = g
    din_ref[...] = din
    xp = f_ref[...] * g
    xn = fp_ref[...] * g
    ta[...] = xp[:, :CW]
    tb[...] = xp[:, CW:]
    tc[...] = xn[:, :CW]
    td[...] = xn[:, CW:]

  return pl.pallas_call(
      body,
      grid=(RB,),
      in_specs=[
          pl.BlockSpec((RBS, FIN), lambda i: (i, 0)),
          pl.BlockSpec((RBS, FIN), lambda i: (i, 0)),
          pl.BlockSpec((RBS, 8), lambda i: (i, 0)),
          pl.BlockSpec((RBS, 8), lambda i: (i, 0)),
      ],
      out_specs=[
          pl.BlockSpec((RBS, 1), lambda i: (i, 0)),
          pl.BlockSpec((RBS, 1), lambda i: (i, 0)),
          pl.BlockSpec((RBS, CW), lambda i: (i, 0)),
          pl.BlockSpec((RBS, CW), lambda i: (i, 0)),
          pl.BlockSpec((RBS, CW), lambda i: (i, 0)),
          pl.BlockSpec((RBS, CW), lambda i: (i, 0)),
      ],
      out_shape=[
          jax.ShapeDtypeStruct((N, 1), jnp.float32),
          jax.ShapeDtypeStruct((N, 1), jnp.float32),
          jax.ShapeDtypeStruct((N, CW), jnp.float32),
          jax.ShapeDtypeStruct((N, CW), jnp.float32),
          jax.ShapeDtypeStruct((N, CW), jnp.float32),
          jax.ShapeDtypeStruct((N, CW), jnp.float32),
      ],
      interpret=interpret,
  )(features, fp, cnt_s, cnt_d)


def _tc3_call(s1, g, din, W1, b1, interpret=False):
  def body(sa, sb, sc, sd, g_ref, din_ref, w_ref, b_ref, *outs):
    din_b = din_ref[...]
    g_b = g_ref[...]
    b = b_ref[...]
    pos = (jnp.dot(sa[...] * din_b, w_ref[:CW, :],
                   preferred_element_type=jnp.float32)
           + jnp.dot(sb[...] * din_b, w_ref[CW:, :],
                     preferred_element_type=jnp.float32) + b)
    neg = (jnp.dot(sc[...] * din_b, w_ref[:CW, :],
                   preferred_element_type=jnp.float32)
           + jnp.dot(sd[...] * din_b, w_ref[CW:, :],
                     preferred_element_type=jnp.float32) + b)
    rp = jnp.maximum(pos, 0.0) * g_b
    rn = jnp.maximum(neg, 0.0) * g_b
    for k in range(4):
      outs[k][...] = rp[:, k * CW:(k + 1) * CW]
      outs[4 + k][...] = rn[:, k * CW:(k + 1) * CW]

  return pl.pallas_call(
      body,
      grid=(RB,),
      in_specs=[
          pl.BlockSpec((RBS, CW), lambda i: (i, 0)),
          pl.BlockSpec((RBS, CW), lambda i: (i, 0)),
          pl.BlockSpec((RBS, CW), lambda i: (i, 0)),
          pl.BlockSpec((RBS, CW), lambda i: (i, 0)),
          pl.BlockSpec((RBS, 1), lambda i: (i, 0)),
          pl.BlockSpec((RBS, 1), lambda i: (i, 0)),
          pl.BlockSpec((FIN, HID), lambda i: (0, 0)),
          pl.BlockSpec((1, HID), lambda i: (0, 0)),
      ],
      out_specs=[pl.BlockSpec((RBS, CW), lambda i: (i, 0))
                 for _ in range(8)],
      out_shape=[jax.ShapeDtypeStruct((N, CW), jnp.float32)
                 for _ in range(8)],
      interpret=interpret,
  )(*s1, g, din, W1, b1)


def _tc4_call(s2, din, W2, b2, ci, interpret=False):
  def body(c0, c1, c2, c3, c4, c5, c6, c7, din_ref, w_ref, b_ref, ci_ref,
           pos_out, neg_out, sum_out, mu_out):
    i = pl.program_id(0)
    din_b = din_ref[...]
    chunks = (c0, c1, c2, c3, c4, c5, c6, c7)
    pos = b_ref[...]
    neg = b_ref[...]
    for k in range(4):
      pos = pos + jnp.dot(chunks[k][...] * din_b,
                          w_ref[k * CW:(k + 1) * CW, :],
                          preferred_element_type=jnp.float32)
      neg = neg + jnp.dot(chunks[4 + k][...] * din_b,
                          w_ref[k * CW:(k + 1) * CW, :],
                          preferred_element_type=jnp.float32)
    pos_out[...] = pos
    neg_out[...] = neg
    nrm = jnp.sqrt(jnp.sum(pos * pos, axis=1, keepdims=True))
    h1 = pos / (nrm + 1e-6)
    rows = i * RBS + lax.broadcasted_iota(jnp.int32, (K, RBS), 1)
    oh = (ci_ref[...] == rows).astype(jnp.float32)
    mu_part = jnp.dot(oh, h1, preferred_element_type=jnp.float32)
    sp = jnp.sum(pos, axis=0, keepdims=True)

    @pl.when(i == 0)
    def _():
      sum_out[...] = sp
      mu_out[...] = mu_part

    @pl.when(i > 0)
    def _():
      sum_out[...] += sp
      mu_out[...] += mu_part

  return pl.pallas_call(
      body,
      grid=(RB,),
      in_specs=[pl.BlockSpec((RBS, CW), lambda i: (i, 0))
                for _ in range(8)] + [
          pl.BlockSpec((RBS, 1), lambda i: (i, 0)),
          pl.BlockSpec((HID, HID), lambda i: (0, 0)),
          pl.BlockSpec((1, HID), lambda i: (0, 0)),
          pl.BlockSpec((K, 1), lambda i: (0, 0)),
      ],
      out_specs=[
          pl.BlockSpec((RBS, HID), lambda i: (i, 0)),
          pl.BlockSpec((RBS, HID), lambda i: (i, 0)),
          pl.BlockSpec((1, HID), lambda i: (0, 0)),
          pl.BlockSpec((K, HID), lambda i: (0, 0)),
      ],
      out_shape=[
          jax.ShapeDtypeStruct((N, HID), jnp.float32),
          jax.ShapeDtypeStruct((N, HID), jnp.float32),
          jax.ShapeDtypeStruct((1, HID), jnp.float32),
          jax.ShapeDtypeStruct((K, HID), jnp.float32),
      ],
      interpret=interpret,
  )(*s2, din, W2, b2, ci)


def _tc5_call(sum_pos, mu_raw, Wd, interpret=False):
  def body(s_ref, m_ref, wd_ref, v_out, mu_out):
    gs = jax.nn.sigmoid(s_ref[...] / N)
    v_out[...] = lax.dot_general(gs, wd_ref[...], (((1,), (1,)), ((), ())),
                                 preferred_element_type=jnp.float32)
    m = m_ref[...]
    mu_out[...] = m / (jnp.sqrt(jnp.sum(m * m, axis=1, keepdims=True)) + 1e-6)

  return pl.pallas_call(
      body,
      out_shape=[
          jax.ShapeDtypeStruct((1, HID), jnp.float32),
          jax.ShapeDtypeStruct((K, HID), jnp.float32),
      ],
      interpret=interpret,
  )(sum_pos, mu_raw, Wd)


def _tc6_call(pos, neg, v, mu, interpret=False):
  def body(pos_ref, neg_ref, v_ref, mu_ref, out_ref):
    i = pl.program_id(0)
    p = pos_ref[...]
    n = neg_ref[...]
    v = v_ref[...]
    mu = mu_ref[...]
    pg = lax.dot_general(p, v, (((1,), (1,)), ((), ())),
                         preferred_element_type=jnp.float32)
    ng = lax.dot_general(n, v, (((1,), (1,)), ((), ())),
                         preferred_element_type=jnp.float32)
    nrm = jnp.sqrt(jnp.sum(p * p, axis=1, keepdims=True))
    h1 = p / (nrm + 1e-6)
    dist = lax.dot_general(h1, mu, (((1,), (1,)), ((), ())),
                           preferred_element_type=jnp.float32)
    z = BETA * dist
    z = z - jnp.max(z, axis=1, keepdims=True)
    ez = jnp.exp(z)
    r = ez / jnp.sum(ez, axis=1, keepdims=True)
    cs = jax.nn.sigmoid(jnp.dot(r, mu, preferred_element_type=jnp.float32))
    pc = jnp.sum(p * cs, axis=1, keepdims=True)
    nc = jnp.sum(n * cs, axis=1, keepdims=True)

    def bce_sum(x, target_one):
      sp = jnp.log1p(jnp.exp(-jnp.abs(x)))
      t = jnp.maximum(x, 0.0) + sp
      if target_one:
        t = t - x
      return jnp.sum(t)

    contrib = (bce_sum(pg, True) + bce_sum(ng, False)
               + ALPHA * (bce_sum(pc, True) + bce_sum(nc, False))) / N

    contrib2d = jnp.reshape(contrib, (1, 1))

    @pl.when(i == 0)
    def _():
      out_ref[...] = contrib2d

    @pl.when(i > 0)
    def _():
      out_ref[...] += contrib2d

  return pl.pallas_call(
      body,
      grid=(RB,),
      in_specs=[
          pl.BlockSpec((RBS, HID), lambda i: (i, 0)),
          pl.BlockSpec((RBS, HID), lambda i: (i, 0)),
          pl.BlockSpec((1, HID), lambda i: (0, 0)),
          pl.BlockSpec((K, HID), lambda i: (0, 0)),
      ],
      out_specs=pl.BlockSpec((1, 1), lambda i: (0, 0)),
      out_shape=jax.ShapeDtypeStruct((1, 1), jnp.float32),
      interpret=interpret,
  )(pos, neg, v, mu)


def _kernel_impl(features, center_index, edge_index, perm, W1, b1, W2, b2, Wd,
                 interpret=False):
  src1 = edge_index[0].astype(jnp.int32)
  dst1 = edge_index[1].astype(jnp.int32)
  src2d = src1.reshape(NBATCH, BB)
  dst2d = dst1.reshape(NBATCH, BB)
  srcp = jnp.concatenate([src1, jnp.zeros((E2 - E,), jnp.int32)]
                         ).reshape(NBATCH2, BB)
  dstp = jnp.concatenate([dst1, jnp.full((E2 - E,), N, jnp.int32)]
                         ).reshape(NBATCH2, BB)
  perm2d = perm.reshape(N // BB, BB).astype(jnp.int32)
  ones8 = jnp.ones((BB, 8), jnp.float32)
  zeros1 = jnp.zeros((RPS, 8), jnp.float32)
  zeros_z = jnp.zeros((RPS, CW), jnp.float32)

  cnt_s, cnt_d, fp = _sc1_call(src2d, dst2d, perm2d, features, ones8, zeros1,
                               interpret=interpret)
  g, din, ta, tb, tc, td = _tc2_call(features, fp, cnt_s, cnt_d,
                                     interpret=interpret)
  s1 = _segsum_call(srcp, dstp, (ta, tb, tc, td), zeros_z,
                    interpret=interpret)
  r8 = _tc3_call(s1, g, din, W1, b1.reshape(1, HID), interpret=interpret)
  s2 = _segsum_call(srcp, dstp, r8, zeros_z, interpret=interpret)
  pos, neg, sum_pos, mu_raw = _tc4_call(
      s2, din, W2, b2.reshape(1, HID),
      center_index.reshape(K, 1).astype(jnp.int32), interpret=interpret)
  v, mu = _tc5_call(sum_pos, mu_raw, Wd, interpret=interpret)
  out = _tc6_call(pos, neg, v, mu, interpret=interpret)
  return out[0, 0]


def kernel(features, center_index, edge_index, perm, W1, b1, W2, b2, Wd):
  return _kernel_impl(features, center_index, edge_index, perm,
                      W1, b1, W2, b2, Wd)


# speedup vs baseline: 7.6107x; 1.0374x over previous
import functools

import jax
import jax.numpy as jnp
from jax import lax
from jax.experimental import pallas as pl
from jax.experimental.pallas import tpu as pltpu
from jax.experimental.pallas import tpu_sc as plsc

N = 10000
E = 160000
FIN = 256
HID = 512
K = 64
BETA = 1.0
ALPHA = 0.5

CW = 128
BB = 80
NBATCH = E // BB
NSUB = 16
BPS = NBATCH // NSUB
BPS3 = 128
NBATCH2 = BPS3 * NSUB
E2 = NBATCH2 * BB
NA = N + 8
RPS = N // NSUB
RB = 10
RBS = N // RB

_mesh_cache = []


def _mesh():
  if not _mesh_cache:
    _mesh_cache.append(
        plsc.VectorSubcoreMesh(core_axis_name="c", subcore_axis_name="s",
                               num_cores=2, num_subcores=16))
  return _mesh_cache[0]


def _sc1_call(src2d, dst2d, perm2d, features, ones8, zeros1, interpret=False):
  def body(src_ref, dst_ref, perm_ref, f_ref, ones_hbm, zer_hbm,
           cs_out, cd_out, fp_out,
           acc_s, acc_d, sidx, didx, pidx, pidx2, ones_v, rows_v, rows_v2,
           sem, sem2):
    c = lax.axis_index("c")
    s = lax.axis_index("s")

    @pl.when(c == 0)
    def _():
      pltpu.sync_copy(zer_hbm, acc_s.at[pl.ds(s * RPS, RPS)])
      pltpu.sync_copy(zer_hbm, acc_d.at[pl.ds(s * RPS, RPS)])
      pltpu.sync_copy(ones_hbm, ones_v)
      pltpu.sync_copy(src_ref.at[pl.ds(s * BPS, BPS)], sidx)
      pltpu.sync_copy(dst_ref.at[pl.ds(s * BPS, BPS)], didx)
      plsc.subcore_barrier()

      def bb(j, carry):
        pltpu.async_copy(ones_v, acc_s.at[sidx.at[j]], sem, add=True)
        pltpu.async_copy(ones_v, acc_d.at[didx.at[j]], sem2, add=True)
        return carry

      lax.fori_loop(0, BPS, bb, 0)

      def dr(j, carry):
        pltpu.make_async_copy(ones_v, acc_s.at[sidx.at[0]], sem).wait()
        pltpu.make_async_copy(ones_v, acc_d.at[didx.at[0]], sem2).wait()
        return carry

      lax.fori_loop(0, BPS, dr, 0)
      plsc.subcore_barrier()
      pltpu.sync_copy(acc_s.at[pl.ds(s * RPS, RPS)],
                      cs_out.at[pl.ds(s * RPS, RPS)])
      pltpu.sync_copy(acc_d.at[pl.ds(s * RPS, RPS)],
                      cd_out.at[pl.ds(s * RPS, RPS)])

    @pl.when(c == 1)
    def _():
      nrow_batch = N // BB
      bufs = (rows_v, rows_v2)
      sems = (sem, sem2)
      pix = (pidx, pidx2)
      for it in range(8):
        b = s + it * NSUB

        @pl.when(b < nrow_batch)
        def _(it=it, b=b):
          pltpu.sync_copy(perm_ref.at[pl.ds(b, 1)], pix[it % 2])
          pltpu.async_copy(f_ref.at[pix[it % 2].at[0]], bufs[it % 2],
                           sems[it % 2])

        if it > 0:
          pb = s + (it - 1) * NSUB

          @pl.when(pb < nrow_batch)
          def _(it=it, pb=pb):
            pltpu.make_async_copy(f_ref.at[pix[(it - 1) % 2].at[0]],
                                  bufs[(it - 1) % 2],
                                  sems[(it - 1) % 2]).wait()
            pltpu.sync_copy(bufs[(it - 1) % 2],
                            fp_out.at[pl.ds(pb * BB, BB)])

      pb = s + 7 * NSUB

      @pl.when(pb < nrow_batch)
      def _(pb=pb):
        pltpu.make_async_copy(f_ref.at[pidx2.at[0]], rows_v2, sem2).wait()
        pltpu.sync_copy(rows_v2, fp_out.at[pl.ds(pb * BB, BB)])

  f = pl.kernel(
      body,
      out_type=[
          jax.ShapeDtypeStruct((N, 8), jnp.float32),
          jax.ShapeDtypeStruct((N, 8), jnp.float32),
          jax.ShapeDtypeStruct((N, FIN), jnp.float32),
      ],
      mesh=_mesh(),
      compiler_params=pltpu.CompilerParams(use_tc_tiling_on_sc=False),
      scratch_types=[
          pltpu.VMEM_SHARED((N, 8), jnp.float32),
          pltpu.VMEM_SHARED((N, 8), jnp.float32),
          pltpu.VMEM((BPS, BB), jnp.int32),
          pltpu.VMEM((BPS, BB), jnp.int32),
          pltpu.VMEM((1, BB), jnp.int32),
          pltpu.VMEM((1, BB), jnp.int32),
          pltpu.VMEM((BB, 8), jnp.float32),
          pltpu.VMEM((BB, FIN), jnp.float32),
          pltpu.VMEM((BB, FIN), jnp.float32),
          pltpu.SemaphoreType.DMA,
          pltpu.SemaphoreType.DMA,
      ],
      interpret=interpret,
  )
  return f(src2d, dst2d, perm2d, features, ones8, zeros1)


def _segsum_call(src2d, dst2d, tables, zeros_z, interpret=False):
  nchunk = len(tables)
  ncpc = nchunk // 2
  dt = tables[0].dtype
  NBUF = 8
  WIN = 4

  def body(src_ref, dst_ref, *rest):
    tabs = rest[:nchunk]
    zer_hbm = rest[nchunk]
    outs = rest[nchunk + 1:2 * nchunk + 1]
    scr = rest[2 * nchunk + 1:]
    acc, sidx, didx = scr[0], scr[1], scr[2]
    rows = scr[3:3 + NBUF]
    gsem = scr[3 + NBUF:3 + 2 * NBUF]
    ssem = scr[3 + 2 * NBUF:3 + 3 * NBUF]
    c = lax.axis_index("c")
    s = lax.axis_index("s")
    pltpu.sync_copy(src_ref.at[pl.ds(s * BPS3, BPS3)], sidx)
    pltpu.sync_copy(dst_ref.at[pl.ds(s * BPS3, BPS3)], didx)

    def one_chunk(tbl, out):
      pltpu.sync_copy(zer_hbm, acc.at[pl.ds(s * RPS, RPS)])
      plsc.subcore_barrier()
      for b in range(WIN):
        pltpu.async_copy(tbl.at[sidx.at[b]], rows[b], gsem[b])

      def grp(g, carry):
        for b in range(NBUF):
          j = 8 * g + b
          nb = (b + WIN) % NBUF
          if b < WIN:
            @pl.when(g > 0)
            def _(nb=nb):
              pltpu.make_async_copy(rows[nb], acc.at[didx.at[0]],
                                    ssem[nb]).wait()
            pltpu.async_copy(tbl.at[sidx.at[j + WIN]], rows[nb], gsem[nb])
          else:
            pltpu.make_async_copy(rows[nb], acc.at[didx.at[0]],
                                  ssem[nb]).wait()

            @pl.when(g < BPS3 // 8 - 1)
            def _(j=j, nb=nb):
              pltpu.async_copy(tbl.at[sidx.at[j + WIN]], rows[nb], gsem[nb])
          pltpu.make_async_copy(tbl.at[sidx.at[0]], rows[b], gsem[b]).wait()
          pltpu.async_copy(rows[b], acc.at[didx.at[j]], ssem[b], add=True)
        return carry

      lax.fori_loop(0, BPS3 // 8, grp, 0)
      for b in range(WIN, NBUF):
        pltpu.make_async_copy(rows[b], acc.at[didx.at[0]], ssem[b]).wait()
      plsc.subcore_barrier()
      pltpu.sync_copy(acc.at[pl.ds(s * RPS, RPS)],
                      out.at[pl.ds(s * RPS, RPS)])

    for ci in range(ncpc):
      for half in range(2):
        idx = half * ncpc + ci

        @pl.when(c == half)
        def _(idx=idx):
          one_chunk(tabs[idx], outs[idx])

  f = pl.kernel(
      body,
      out_type=[jax.ShapeDtypeStruct((N, CW), dt)
                for _ in range(nchunk)],
      mesh=_mesh(),
      compiler_params=pltpu.CompilerParams(use_tc_tiling_on_sc=False),
      scratch_types=(
          [pltpu.VMEM_SHARED((NA, CW), dt),
           pltpu.VMEM((BPS3, BB), jnp.int32),
           pltpu.VMEM((BPS3, BB), jnp.int32)]
          + [pltpu.VMEM((BB, CW), dt) for _ in range(NBUF)]
          + [pltpu.SemaphoreType.DMA for _ in range(2 * NBUF)]
      ),
      interpret=interpret,
  )
  return f(src2d, dst2d, *tables, zeros_z)


def _tc2_call(features, fp, cnt_s, cnt_d, interpret=False):
  def body(f_ref, fp_ref, cs_ref, cd_ref, g_ref, din_ref, ta, tb, tc, td):
    deg_o = jnp.maximum(cs_ref[:, :1], 1.0)
    deg_i = jnp.maximum(cd_ref[:, :1], 1.0)
    g = lax.rsqrt(deg_o)
    din = lax.rsqrt(deg_i)
    g_ref[...] = g
    din_ref[...] = din
    xp = f_ref[...] * g
    xn = fp_ref[...] * g
    xpb = xp.astype(jnp.bfloat16)
    xnb = xn.astype(jnp.bfloat16)
    ta[...] = xpb[:, :CW]
    tb[...] = xpb[:, CW:]
    tc[...] = xnb[:, :CW]
    td[...] = xnb[:, CW:]

  return pl.pallas_call(
      body,
      grid=(RB,),
      in_specs=[
          pl.BlockSpec((RBS, FIN), lambda i: (i, 0)),
          pl.BlockSpec((RBS, FIN), lambda i: (i, 0)),
          pl.BlockSpec((RBS, 8), lambda i: (i, 0)),
          pl.BlockSpec((RBS, 8), lambda i: (i, 0)),
      ],
      out_specs=[
          pl.BlockSpec((RBS, 1), lambda i: (i, 0)),
          pl.BlockSpec((RBS, 1), lambda i: (i, 0)),
          pl.BlockSpec((RBS, CW), lambda i: (i, 0)),
          pl.BlockSpec((RBS, CW), lambda i: (i, 0)),
          pl.BlockSpec((RBS, CW), lambda i: (i, 0)),
          pl.BlockSpec((RBS, CW), lambda i: (i, 0)),
      ],
      out_shape=[
          jax.ShapeDtypeStruct((N, 1), jnp.float32),
          jax.ShapeDtypeStruct((N, 1), jnp.float32),
          jax.ShapeDtypeStruct((N, CW), jnp.bfloat16),
          jax.ShapeDtypeStruct((N, CW), jnp.bfloat16),
          jax.ShapeDtypeStruct((N, CW), jnp.bfloat16),
          jax.ShapeDtypeStruct((N, CW), jnp.bfloat16),
      ],
      interpret=interpret,
  )(features, fp, cnt_s, cnt_d)


def _tc3_call(s1, g, din, W1, b1, interpret=False):
  def body(sa, sb, sc, sd, g_ref, din_ref, w_ref, b_ref, *outs):
    din_b = din_ref[...]
    g_b = g_ref[...]
    b = b_ref[...]
    pos = (jnp.dot(sa[...].astype(jnp.float32) * din_b, w_ref[:CW, :],
                   preferred_element_type=jnp.float32)
           + jnp.dot(sb[...].astype(jnp.float32) * din_b, w_ref[CW:, :],
                     preferred_element_type=jnp.float32) + b)
    neg = (jnp.dot(sc[...].astype(jnp.float32) * din_b, w_ref[:CW, :],
                   preferred_element_type=jnp.float32)
           + jnp.dot(sd[...].astype(jnp.float32) * din_b, w_ref[CW:, :],
                     preferred_element_type=jnp.float32) + b)
    rp = (jnp.maximum(pos, 0.0) * g_b).astype(jnp.bfloat16)
    rn = (jnp.maximum(neg, 0.0) * g_b).astype(jnp.bfloat16)
    for k in range(4):
      outs[k][...] = rp[:, k * CW:(k + 1) * CW]
      outs[4 + k][...] = rn[:, k * CW:(k + 1) * CW]

  return pl.pallas_call(
      body,
      grid=(RB,),
      in_specs=[
          pl.BlockSpec((RBS, CW), lambda i: (i, 0)),
          pl.BlockSpec((RBS, CW), lambda i: (i, 0)),
          pl.BlockSpec((RBS, CW), lambda i: (i, 0)),
          pl.BlockSpec((RBS, CW), lambda i: (i, 0)),
          pl.BlockSpec((RBS, 1), lambda i: (i, 0)),
          pl.BlockSpec((RBS, 1), lambda i: (i, 0)),
          pl.BlockSpec((FIN, HID), lambda i: (0, 0)),
          pl.BlockSpec((1, HID), lambda i: (0, 0)),
      ],
      out_specs=[pl.BlockSpec((RBS, CW), lambda i: (i, 0))
                 for _ in range(8)],
      out_shape=[jax.ShapeDtypeStruct((N, CW), jnp.bfloat16)
                 for _ in range(8)],
      interpret=interpret,
  )(*s1, g, din, W1, b1)


def _tc4_call(s2, din, W2, b2, ci, interpret=False):
  def body(c0, c1, c2, c3, c4, c5, c6, c7, din_ref, w_ref, b_ref, ci_ref,
           pos_out, neg_out, sum_out, mu_out):
    i = pl.program_id(0)
    din_b = din_ref[...]
    chunks = (c0, c1, c2, c3, c4, c5, c6, c7)
    pos = b_ref[...]
    neg = b_ref[...]
    for k in range(4):
      pos = pos + jnp.dot(chunks[k][...].astype(jnp.float32) * din_b,
                          w_ref[k * CW:(k + 1) * CW, :],
                          preferred_element_type=jnp.float32)
      neg = neg + jnp.dot(chunks[4 + k][...].astype(jnp.float32) * din_b,
                          w_ref[k * CW:(k + 1) * CW, :],
                          preferred_element_type=jnp.float32)
    pos_out[...] = pos
    neg_out[...] = neg
    nrm = jnp.sqrt(jnp.sum(pos * pos, axis=1, keepdims=True))
    h1 = pos / (nrm + 1e-6)
    rows = i * RBS + lax.broadcasted_iota(jnp.int32, (K, RBS), 1)
    oh = (ci_ref[...] == rows).astype(jnp.float32)
    mu_part = jnp.dot(oh, h1, preferred_element_type=jnp.float32)
    sp = jnp.sum(pos, axis=0, keepdims=True)

    @pl.when(i == 0)
    def _():
      sum_out[...] = sp
      mu_out[...] = mu_part

    @pl.when(i > 0)
    def _():
      sum_out[...] += sp
      mu_out[...] += mu_part

  return pl.pallas_call(
      body,
      grid=(RB,),
      in_specs=[pl.BlockSpec((RBS, CW), lambda i: (i, 0))
                for _ in range(8)] + [
          pl.BlockSpec((RBS, 1), lambda i: (i, 0)),
          pl.BlockSpec((HID, HID), lambda i: (0, 0)),
          pl.BlockSpec((1, HID), lambda i: (0, 0)),
          pl.BlockSpec((K, 1), lambda i: (0, 0)),
      ],
      out_specs=[
          pl.BlockSpec((RBS, HID), lambda i: (i, 0)),
          pl.BlockSpec((RBS, HID), lambda i: (i, 0)),
          pl.BlockSpec((1, HID), lambda i: (0, 0)),
          pl.BlockSpec((K, HID), lambda i: (0, 0)),
      ],
      out_shape=[
          jax.ShapeDtypeStruct((N, HID), jnp.float32),
          jax.ShapeDtypeStruct((N, HID), jnp.float32),
          jax.ShapeDtypeStruct((1, HID), jnp.float32),
          jax.ShapeDtypeStruct((K, HID), jnp.float32),
      ],
      interpret=interpret,
  )(*s2, din, W2, b2, ci)


def _tc5_call(sum_pos, mu_raw, Wd, interpret=False):
  def body(s_ref, m_ref, wd_ref, v_out, mu_out):
    gs = jax.nn.sigmoid(s_ref[...] / N)
    v_out[...] = lax.dot_general(gs, wd_ref[...], (((1,), (1,)), ((), ())),
                                 preferred_element_type=jnp.float32)
    m = m_ref[...]
    mu_out[...] = m / (jnp.sqrt(jnp.sum(m * m, axis=1, keepdims=True)) + 1e-6)

  return pl.pallas_call(
      body,
      out_shape=[
          jax.ShapeDtypeStruct((1, HID), jnp.float32),
          jax.ShapeDtypeStruct((K, HID), jnp.float32),
      ],
      interpret=interpret,
  )(sum_pos, mu_raw, Wd)


def _tc6_call(pos, neg, v, mu, interpret=False):
  def body(pos_ref, neg_ref, v_ref, mu_ref, out_ref):
    i = pl.program_id(0)
    p = pos_ref[...]
    n = neg_ref[...]
    v = v_ref[...]
    mu = mu_ref[...]
    pg = lax.dot_general(p, v, (((1,), (1,)), ((), ())),
                         preferred_element_type=jnp.float32)
    ng = lax.dot_general(n, v, (((1,), (1,)), ((), ())),
                         preferred_element_type=jnp.float32)
    nrm = jnp.sqrt(jnp.sum(p * p, axis=1, keepdims=True))
    h1 = p / (nrm + 1e-6)
    dist = lax.dot_general(h1, mu, (((1,), (1,)), ((), ())),
                           preferred_element_type=jnp.float32)
    z = BETA * dist
    z = z - jnp.max(z, axis=1, keepdims=True)
    ez = jnp.exp(z)
    r = ez / jnp.sum(ez, axis=1, keepdims=True)
    cs = jax.nn.sigmoid(jnp.dot(r, mu, preferred_element_type=jnp.float32))
    pc = jnp.sum(p * cs, axis=1, keepdims=True)
    nc = jnp.sum(n * cs, axis=1, keepdims=True)

    def bce_sum(x, target_one):
      sp = jnp.log1p(jnp.exp(-jnp.abs(x)))
      t = jnp.maximum(x, 0.0) + sp
      if target_one:
        t = t - x
      return jnp.sum(t)

    contrib = (bce_sum(pg, True) + bce_sum(ng, False)
               + ALPHA * (bce_sum(pc, True) + bce_sum(nc, False))) / N

    contrib2d = jnp.reshape(contrib, (1, 1))

    @pl.when(i == 0)
    def _():
      out_ref[...] = contrib2d

    @pl.when(i > 0)
    def _():
      out_ref[...] += contrib2d

  return pl.pallas_call(
      body,
      grid=(RB,),
      in_specs=[
          pl.BlockSpec((RBS, HID), lambda i: (i, 0)),
          pl.BlockSpec((RBS, HID), lambda i: (i, 0)),
          pl.BlockSpec((1, HID), lambda i: (0, 0)),
          pl.BlockSpec((K, HID), lambda i: (0, 0)),
      ],
      out_specs=pl.BlockSpec((1, 1), lambda i: (0, 0)),
      out_shape=jax.ShapeDtypeStruct((1, 1), jnp.float32),
      interpret=interpret,
  )(pos, neg, v, mu)


def _kernel_impl(features, center_index, edge_index, perm, W1, b1, W2, b2, Wd,
                 interpret=False):
  src1 = edge_index[0].astype(jnp.int32)
  dst1 = edge_index[1].astype(jnp.int32)
  src2d = src1.reshape(NBATCH, BB)
  dst2d = dst1.reshape(NBATCH, BB)
  srcp = jnp.concatenate([src1, jnp.zeros((E2 - E,), jnp.int32)]
                         ).reshape(NBATCH2, BB)
  dstp = jnp.concatenate([dst1, jnp.full((E2 - E,), N, jnp.int32)]
                         ).reshape(NBATCH2, BB)
  perm2d = perm.reshape(N // BB, BB).astype(jnp.int32)
  ones8 = jnp.ones((BB, 8), jnp.float32)
  zeros1 = jnp.zeros((RPS, 8), jnp.float32)
  zeros_z = jnp.zeros((RPS, CW), jnp.bfloat16)

  cnt_s, cnt_d, fp = _sc1_call(src2d, dst2d, perm2d, features, ones8, zeros1,
                               interpret=interpret)
  g, din, ta, tb, tc, td = _tc2_call(features, fp, cnt_s, cnt_d,
                                     interpret=interpret)
  s1 = _segsum_call(srcp, dstp, (ta, tb, tc, td), zeros_z,
                    interpret=interpret)
  r8 = _tc3_call(s1, g, din, W1, b1.reshape(1, HID), interpret=interpret)
  s2 = _segsum_call(srcp, dstp, r8, zeros_z, interpret=interpret)
  pos, neg, sum_pos, mu_raw = _tc4_call(
      s2, din, W2, b2.reshape(1, HID),
      center_index.reshape(K, 1).astype(jnp.int32), interpret=interpret)
  v, mu = _tc5_call(sum_pos, mu_raw, Wd, interpret=interpret)
  out = _tc6_call(pos, neg, v, mu, interpret=interpret)
  return out[0, 0]


def kernel(features, center_index, edge_index, perm, W1, b1, W2, b2, Wd):
  return _kernel_impl(features, center_index, edge_index, perm,
                      W1, b1, W2, b2, Wd)


# speedup vs baseline: 7.6681x; 1.0075x over previous
import functools

import jax
import jax.numpy as jnp
from jax import lax
from jax.experimental import pallas as pl
from jax.experimental.pallas import tpu as pltpu
from jax.experimental.pallas import tpu_sc as plsc

N = 10000
E = 160000
FIN = 256
HID = 512
K = 64
BETA = 1.0
ALPHA = 0.5

CW = 128
BB = 80
NBATCH = E // BB
NSUB = 16
BPS = NBATCH // NSUB
BPS3 = 128
NBATCH2 = BPS3 * NSUB
E2 = NBATCH2 * BB
NA = N + 8
RPS = N // NSUB
RB = 10
RBS = N // RB

_mesh_cache = []


def _mesh():
  if not _mesh_cache:
    _mesh_cache.append(
        plsc.VectorSubcoreMesh(core_axis_name="c", subcore_axis_name="s",
                               num_cores=2, num_subcores=16))
  return _mesh_cache[0]


def _sc1_call(src2d, dst2d, perm2d, features, ones8, zeros1, interpret=False):
  def body(src_ref, dst_ref, perm_ref, f_ref, ones_hbm, zer_hbm,
           cs_out, cd_out, fp_out,
           acc_s, acc_d, sidx, didx, pidx, pidx2, ones_v, rows_v, rows_v2,
           sem, sem2):
    c = lax.axis_index("c")
    s = lax.axis_index("s")

    @pl.when(c == 0)
    def _():
      pltpu.sync_copy(zer_hbm, acc_s.at[pl.ds(s * RPS, RPS)])
      pltpu.sync_copy(zer_hbm, acc_d.at[pl.ds(s * RPS, RPS)])
      pltpu.sync_copy(ones_hbm, ones_v)
      pltpu.sync_copy(src_ref.at[pl.ds(s * BPS, BPS)], sidx)
      pltpu.sync_copy(dst_ref.at[pl.ds(s * BPS, BPS)], didx)
      plsc.subcore_barrier()

      def bb(j, carry):
        pltpu.async_copy(ones_v, acc_s.at[sidx.at[j]], sem, add=True)
        pltpu.async_copy(ones_v, acc_d.at[didx.at[j]], sem2, add=True)
        return carry

      lax.fori_loop(0, BPS, bb, 0)

      def dr(j, carry):
        pltpu.make_async_copy(ones_v, acc_s.at[sidx.at[0]], sem).wait()
        pltpu.make_async_copy(ones_v, acc_d.at[didx.at[0]], sem2).wait()
        return carry

      lax.fori_loop(0, BPS, dr, 0)
      plsc.subcore_barrier()
      pltpu.sync_copy(acc_s.at[pl.ds(s * RPS, RPS)],
                      cs_out.at[pl.ds(s * RPS, RPS)])
      pltpu.sync_copy(acc_d.at[pl.ds(s * RPS, RPS)],
                      cd_out.at[pl.ds(s * RPS, RPS)])

    @pl.when(c == 1)
    def _():
      nrow_batch = N // BB
      bufs = (rows_v, rows_v2)
      sems = (sem, sem2)
      pix = (pidx, pidx2)
      for it in range(8):
        b = s + it * NSUB

        @pl.when(b < nrow_batch)
        def _(it=it, b=b):
          pltpu.sync_copy(perm_ref.at[pl.ds(b, 1)], pix[it % 2])
          pltpu.async_copy(f_ref.at[pix[it % 2].at[0]], bufs[it % 2],
                           sems[it % 2])

        if it > 0:
          pb = s + (it - 1) * NSUB

          @pl.when(pb < nrow_batch)
          def _(it=it, pb=pb):
            pltpu.make_async_copy(f_ref.at[pix[(it - 1) % 2].at[0]],
                                  bufs[(it - 1) % 2],
                                  sems[(it - 1) % 2]).wait()
            pltpu.sync_copy(bufs[(it - 1) % 2],
                            fp_out.at[pl.ds(pb * BB, BB)])

      pb = s + 7 * NSUB

      @pl.when(pb < nrow_batch)
      def _(pb=pb):
        pltpu.make_async_copy(f_ref.at[pidx2.at[0]], rows_v2, sem2).wait()
        pltpu.sync_copy(rows_v2, fp_out.at[pl.ds(pb * BB, BB)])

  f = pl.kernel(
      body,
      out_type=[
          jax.ShapeDtypeStruct((N, 8), jnp.float32),
          jax.ShapeDtypeStruct((N, 8), jnp.float32),
          jax.ShapeDtypeStruct((N, FIN), jnp.float32),
      ],
      mesh=_mesh(),
      compiler_params=pltpu.CompilerParams(use_tc_tiling_on_sc=False),
      scratch_types=[
          pltpu.VMEM_SHARED((N, 8), jnp.float32),
          pltpu.VMEM_SHARED((N, 8), jnp.float32),
          pltpu.VMEM((BPS, BB), jnp.int32),
          pltpu.VMEM((BPS, BB), jnp.int32),
          pltpu.VMEM((1, BB), jnp.int32),
          pltpu.VMEM((1, BB), jnp.int32),
          pltpu.VMEM((BB, 8), jnp.float32),
          pltpu.VMEM((BB, FIN), jnp.float32),
          pltpu.VMEM((BB, FIN), jnp.float32),
          pltpu.SemaphoreType.DMA,
          pltpu.SemaphoreType.DMA,
      ],
      interpret=interpret,
  )
  return f(src2d, dst2d, perm2d, features, ones8, zeros1)


def _segsum_call(src2d, dst2d, tables, zeros_z, interpret=False):
  nchunk = len(tables)
  ncpc = nchunk // 2
  dt = tables[0].dtype
  NBUF = 8
  WIN = 4
  SBB = 128
  SBPS = E2 // (SBB * NSUB)

  def body(src_ref, dst_ref, *rest):
    tabs = rest[:nchunk]
    zer_hbm = rest[nchunk]
    outs = rest[nchunk + 1:2 * nchunk + 1]
    scr = rest[2 * nchunk + 1:]
    acc, sidx, didx = scr[0], scr[1], scr[2]
    rows = scr[3:3 + NBUF]
    gsem = scr[3 + NBUF:3 + 2 * NBUF]
    ssem = scr[3 + 2 * NBUF:3 + 3 * NBUF]
    c = lax.axis_index("c")
    s = lax.axis_index("s")
    pltpu.sync_copy(src_ref.at[pl.ds(s * SBPS, SBPS)], sidx)
    pltpu.sync_copy(dst_ref.at[pl.ds(s * SBPS, SBPS)], didx)

    def one_chunk(tbl, out):
      pltpu.sync_copy(zer_hbm, acc.at[pl.ds(s * RPS, RPS)])
      plsc.subcore_barrier()
      for b in range(WIN):
        pltpu.async_copy(tbl.at[sidx.at[b]], rows[b], gsem[b])

      def grp(g, carry):
        for b in range(NBUF):
          j = 8 * g + b
          nb = (b + WIN) % NBUF
          if b < WIN:
            @pl.when(g > 0)
            def _(nb=nb):
              pltpu.make_async_copy(rows[nb], acc.at[didx.at[0]],
                                    ssem[nb]).wait()
            pltpu.async_copy(tbl.at[sidx.at[j + WIN]], rows[nb], gsem[nb])
          else:
            pltpu.make_async_copy(rows[nb], acc.at[didx.at[0]],
                                  ssem[nb]).wait()

            @pl.when(g < SBPS // 8 - 1)
            def _(j=j, nb=nb):
              pltpu.async_copy(tbl.at[sidx.at[j + WIN]], rows[nb], gsem[nb])
          pltpu.make_async_copy(tbl.at[sidx.at[0]], rows[b], gsem[b]).wait()
          pltpu.async_copy(rows[b], acc.at[didx.at[j]], ssem[b], add=True)
        return carry

      lax.fori_loop(0, SBPS // 8, grp, 0)
      for b in range(WIN, NBUF):
        pltpu.make_async_copy(rows[b], acc.at[didx.at[0]], ssem[b]).wait()
      plsc.subcore_barrier()
      pltpu.sync_copy(acc.at[pl.ds(s * RPS, RPS)],
                      out.at[pl.ds(s * RPS, RPS)])

    for ci in range(ncpc):
      for half in range(2):
        idx = half * ncpc + ci

        @pl.when(c == half)
        def _(idx=idx):
          one_chunk(tabs[idx], outs[idx])

  f = pl.kernel(
      body,
      out_type=[jax.ShapeDtypeStruct((N, CW), dt)
                for _ in range(nchunk)],
      mesh=_mesh(),
      compiler_params=pltpu.CompilerParams(use_tc_tiling_on_sc=False),
      scratch_types=(
          [pltpu.VMEM_SHARED((NA, CW), dt),
           pltpu.VMEM((SBPS, SBB), jnp.int32),
           pltpu.VMEM((SBPS, SBB), jnp.int32)]
          + [pltpu.VMEM((SBB, CW), dt) for _ in range(NBUF)]
          + [pltpu.SemaphoreType.DMA for _ in range(2 * NBUF)]
      ),
      interpret=interpret,
  )
  return f(src2d, dst2d, *tables, zeros_z)


def _tc2_call(features, fp, cnt_s, cnt_d, interpret=False):
  def body(f_ref, fp_ref, cs_ref, cd_ref, g_ref, din_ref, ta, tb, tc, td):
    deg_o = jnp.maximum(cs_ref[:, :1], 1.0)
    deg_i = jnp.maximum(cd_ref[:, :1], 1.0)
    g = lax.rsqrt(deg_o)
    din = lax.rsqrt(deg_i)
    g_ref[...] = g
    din_ref[...] = din
    xp = f_ref[...] * g
    xn = fp_ref[...] * g
    xpb = xp.astype(jnp.bfloat16)
    xnb = xn.astype(jnp.bfloat16)
    ta[...] = xpb[:, :CW]
    tb[...] = xpb[:, CW:]
    tc[...] = xnb[:, :CW]
    td[...] = xnb[:, CW:]

  return pl.pallas_call(
      body,
      grid=(RB,),
      in_specs=[
          pl.BlockSpec((RBS, FIN), lambda i: (i, 0)),
          pl.BlockSpec((RBS, FIN), lambda i: (i, 0)),
          pl.BlockSpec((RBS, 8), lambda i: (i, 0)),
          pl.BlockSpec((RBS, 8), lambda i: (i, 0)),
      ],
      out_specs=[
          pl.BlockSpec((RBS, 1), lambda i: (i, 0)),
          pl.BlockSpec((RBS, 1), lambda i: (i, 0)),
          pl.BlockSpec((RBS, CW), lambda i: (i, 0)),
          pl.BlockSpec((RBS, CW), lambda i: (i, 0)),
          pl.BlockSpec((RBS, CW), lambda i: (i, 0)),
          pl.BlockSpec((RBS, CW), lambda i: (i, 0)),
      ],
      out_shape=[
          jax.ShapeDtypeStruct((N, 1), jnp.float32),
          jax.ShapeDtypeStruct((N, 1), jnp.float32),
          jax.ShapeDtypeStruct((N, CW), jnp.bfloat16),
          jax.ShapeDtypeStruct((N, CW), jnp.bfloat16),
          jax.ShapeDtypeStruct((N, CW), jnp.bfloat16),
          jax.ShapeDtypeStruct((N, CW), jnp.bfloat16),
      ],
      interpret=interpret,
  )(features, fp, cnt_s, cnt_d)


def _tc3_call(s1, g, din, W1, b1, interpret=False):
  def body(sa, sb, sc, sd, g_ref, din_ref, w_ref, b_ref, *outs):
    din_b = din_ref[...]
    g_b = g_ref[...]
    b = b_ref[...]
    pos = (jnp.dot(sa[...].astype(jnp.float32) * din_b, w_ref[:CW, :],
                   preferred_element_type=jnp.float32)
           + jnp.dot(sb[...].astype(jnp.float32) * din_b, w_ref[CW:, :],
                     preferred_element_type=jnp.float32) + b)
    neg = (jnp.dot(sc[...].astype(jnp.float32) * din_b, w_ref[:CW, :],
                   preferred_element_type=jnp.float32)
           + jnp.dot(sd[...].astype(jnp.float32) * din_b, w_ref[CW:, :],
                     preferred_element_type=jnp.float32) + b)
    rp = (jnp.maximum(pos, 0.0) * g_b).astype(jnp.bfloat16)
    rn = (jnp.maximum(neg, 0.0) * g_b).astype(jnp.bfloat16)
    for k in range(4):
      outs[k][...] = rp[:, k * CW:(k + 1) * CW]
      outs[4 + k][...] = rn[:, k * CW:(k + 1) * CW]

  return pl.pallas_call(
      body,
      grid=(RB,),
      in_specs=[
          pl.BlockSpec((RBS, CW), lambda i: (i, 0)),
          pl.BlockSpec((RBS, CW), lambda i: (i, 0)),
          pl.BlockSpec((RBS, CW), lambda i: (i, 0)),
          pl.BlockSpec((RBS, CW), lambda i: (i, 0)),
          pl.BlockSpec((RBS, 1), lambda i: (i, 0)),
          pl.BlockSpec((RBS, 1), lambda i: (i, 0)),
          pl.BlockSpec((FIN, HID), lambda i: (0, 0)),
          pl.BlockSpec((1, HID), lambda i: (0, 0)),
      ],
      out_specs=[pl.BlockSpec((RBS, CW), lambda i: (i, 0))
                 for _ in range(8)],
      out_shape=[jax.ShapeDtypeStruct((N, CW), jnp.bfloat16)
                 for _ in range(8)],
      interpret=interpret,
  )(*s1, g, din, W1, b1)


def _tc4_call(s2, din, W2, b2, ci, interpret=False):
  def body(c0, c1, c2, c3, c4, c5, c6, c7, din_ref, w_ref, b_ref, ci_ref,
           pos_out, neg_out, sum_out, mu_out):
    i = pl.program_id(0)
    din_b = din_ref[...]
    chunks = (c0, c1, c2, c3, c4, c5, c6, c7)
    pos = b_ref[...]
    neg = b_ref[...]
    for k in range(4):
      pos = pos + jnp.dot(chunks[k][...].astype(jnp.float32) * din_b,
                          w_ref[k * CW:(k + 1) * CW, :],
                          preferred_element_type=jnp.float32)
      neg = neg + jnp.dot(chunks[4 + k][...].astype(jnp.float32) * din_b,
                          w_ref[k * CW:(k + 1) * CW, :],
                          preferred_element_type=jnp.float32)
    pos_out[...] = pos
    neg_out[...] = neg
    nrm = jnp.sqrt(jnp.sum(pos * pos, axis=1, keepdims=True))
    h1 = pos / (nrm + 1e-6)
    rows = i * RBS + lax.broadcasted_iota(jnp.int32, (K, RBS), 1)
    oh = (ci_ref[...] == rows).astype(jnp.float32)
    mu_part = jnp.dot(oh, h1, preferred_element_type=jnp.float32)
    sp = jnp.sum(pos, axis=0, keepdims=True)

    @pl.when(i == 0)
    def _():
      sum_out[...] = sp
      mu_out[...] = mu_part

    @pl.when(i > 0)
    def _():
      sum_out[...] += sp
      mu_out[...] += mu_part

  return pl.pallas_call(
      body,
      grid=(RB,),
      in_specs=[pl.BlockSpec((RBS, CW), lambda i: (i, 0))
                for _ in range(8)] + [
          pl.BlockSpec((RBS, 1), lambda i: (i, 0)),
          pl.BlockSpec((HID, HID), lambda i: (0, 0)),
          pl.BlockSpec((1, HID), lambda i: (0, 0)),
          pl.BlockSpec((K, 1), lambda i: (0, 0)),
      ],
      out_specs=[
          pl.BlockSpec((RBS, HID), lambda i: (i, 0)),
          pl.BlockSpec((RBS, HID), lambda i: (i, 0)),
          pl.BlockSpec((1, HID), lambda i: (0, 0)),
          pl.BlockSpec((K, HID), lambda i: (0, 0)),
      ],
      out_shape=[
          jax.ShapeDtypeStruct((N, HID), jnp.float32),
          jax.ShapeDtypeStruct((N, HID), jnp.float32),
          jax.ShapeDtypeStruct((1, HID), jnp.float32),
          jax.ShapeDtypeStruct((K, HID), jnp.float32),
      ],
      interpret=interpret,
  )(*s2, din, W2, b2, ci)


def _tc5_call(sum_pos, mu_raw, Wd, interpret=False):
  def body(s_ref, m_ref, wd_ref, v_out, mu_out):
    gs = jax.nn.sigmoid(s_ref[...] / N)
    v_out[...] = lax.dot_general(gs, wd_ref[...], (((1,), (1,)), ((), ())),
                                 preferred_element_type=jnp.float32)
    m = m_ref[...]
    mu_out[...] = m / (jnp.sqrt(jnp.sum(m * m, axis=1, keepdims=True)) + 1e-6)

  return pl.pallas_call(
      body,
      out_shape=[
          jax.ShapeDtypeStruct((1, HID), jnp.float32),
          jax.ShapeDtypeStruct((K, HID), jnp.float32),
      ],
      interpret=interpret,
  )(sum_pos, mu_raw, Wd)


def _tc6_call(pos, neg, v, mu, interpret=False):
  def body(pos_ref, neg_ref, v_ref, mu_ref, out_ref):
    i = pl.program_id(0)
    p = pos_ref[...]
    n = neg_ref[...]
    v = v_ref[...]
    mu = mu_ref[...]
    pg = lax.dot_general(p, v, (((1,), (1,)), ((), ())),
                         preferred_element_type=jnp.float32)
    ng = lax.dot_general(n, v, (((1,), (1,)), ((), ())),
                         preferred_element_type=jnp.float32)
    nrm = jnp.sqrt(jnp.sum(p * p, axis=1, keepdims=True))
    h1 = p / (nrm + 1e-6)
    dist = lax.dot_general(h1, mu, (((1,), (1,)), ((), ())),
                           preferred_element_type=jnp.float32)
    z = BETA * dist
    z = z - jnp.max(z, axis=1, keepdims=True)
    ez = jnp.exp(z)
    r = ez / jnp.sum(ez, axis=1, keepdims=True)
    cs = jax.nn.sigmoid(jnp.dot(r, mu, preferred_element_type=jnp.float32))
    pc = jnp.sum(p * cs, axis=1, keepdims=True)
    nc = jnp.sum(n * cs, axis=1, keepdims=True)

    def bce_sum(x, target_one):
      sp = jnp.log1p(jnp.exp(-jnp.abs(x)))
      t = jnp.maximum(x, 0.0) + sp
      if target_one:
        t = t - x
      return jnp.sum(t)

    contrib = (bce_sum(pg, True) + bce_sum(ng, False)
               + ALPHA * (bce_sum(pc, True) + bce_sum(nc, False))) / N

    contrib2d = jnp.reshape(contrib, (1, 1))

    @pl.when(i == 0)
    def _():
      out_ref[...] = contrib2d

    @pl.when(i > 0)
    def _():
      out_ref[...] += contrib2d

  return pl.pallas_call(
      body,
      grid=(RB,),
      in_specs=[
          pl.BlockSpec((RBS, HID), lambda i: (i, 0)),
          pl.BlockSpec((RBS, HID), lambda i: (i, 0)),
          pl.BlockSpec((1, HID), lambda i: (0, 0)),
          pl.BlockSpec((K, HID), lambda i: (0, 0)),
      ],
      out_specs=pl.BlockSpec((1, 1), lambda i: (0, 0)),
      out_shape=jax.ShapeDtypeStruct((1, 1), jnp.float32),
      interpret=interpret,
  )(pos, neg, v, mu)


def _kernel_impl(features, center_index, edge_index, perm, W1, b1, W2, b2, Wd,
                 interpret=False):
  src1 = edge_index[0].astype(jnp.int32)
  dst1 = edge_index[1].astype(jnp.int32)
  src2d = src1.reshape(NBATCH, BB)
  dst2d = dst1.reshape(NBATCH, BB)
  srcp = jnp.concatenate([src1, jnp.zeros((E2 - E,), jnp.int32)]
                         ).reshape(E2 // 128, 128)
  dstp = jnp.concatenate([dst1, jnp.full((E2 - E,), N, jnp.int32)]
                         ).reshape(E2 // 128, 128)
  perm2d = perm.reshape(N // BB, BB).astype(jnp.int32)
  ones8 = jnp.ones((BB, 8), jnp.float32)
  zeros1 = jnp.zeros((RPS, 8), jnp.float32)
  zeros_z = jnp.zeros((RPS, CW), jnp.bfloat16)

  cnt_s, cnt_d, fp = _sc1_call(src2d, dst2d, perm2d, features, ones8, zeros1,
                               interpret=interpret)
  g, din, ta, tb, tc, td = _tc2_call(features, fp, cnt_s, cnt_d,
                                     interpret=interpret)
  s1 = _segsum_call(srcp, dstp, (ta, tb, tc, td), zeros_z,
                    interpret=interpret)
  r8 = _tc3_call(s1, g, din, W1, b1.reshape(1, HID), interpret=interpret)
  s2 = _segsum_call(srcp, dstp, r8, zeros_z, interpret=interpret)
  pos, neg, sum_pos, mu_raw = _tc4_call(
      s2, din, W2, b2.reshape(1, HID),
      center_index.reshape(K, 1).astype(jnp.int32), interpret=interpret)
  v, mu = _tc5_call(sum_pos, mu_raw, Wd, interpret=interpret)
  out = _tc6_call(pos, neg, v, mu, interpret=interpret)
  return out[0, 0]


def kernel(features, center_index, edge_index, perm, W1, b1, W2, b2, Wd):
  return _kernel_impl(features, center_index, edge_index, perm,
                      W1, b1, W2, b2, Wd)
